# Initial kernel scaffold; baseline (speedup 1.0000x reference)
import functools

import jax
import jax.numpy as jnp
from jax import lax
from jax.experimental import pallas as pl
from jax.experimental.pallas import tpu as pltpu
from jax.experimental.pallas import tpu_sc as plsc

EMBED = 256
HIDDEN = 1024
HEADS = 8
DH = EMBED // HEADS
HH = HEADS // 2
HW = EMBED // 2
L = 10000
E = 160000
INV_SQRT_DH = 1.0 / (DH ** 0.5)

ROWS_BLK = 1000
GRID = L // ROWS_BLK

NSUB = 16
CHUNK = 128
NCHUNK = E // CHUNK
ACC_W = 144
STRIPE = L // NSUB
SUBSTRIPE = 125


def _ln(x, g, b, eps=1e-5):
    mu = jnp.mean(x, axis=-1, keepdims=True)
    var = jnp.mean((x - mu) ** 2, axis=-1, keepdims=True)
    return (x - mu) * lax.rsqrt(var + eps) * g + b


def _proj_body(x_ref, wq_ref, bq_ref, wk_ref, bk_ref, wv_ref, bv_ref,
               g_ref, b_ref, q_ref, k_ref, v_ref):
    z = _ln(x_ref[:], g_ref[:], b_ref[:])
    for w_ref, bias_ref, o_ref in ((wq_ref, bq_ref, q_ref),
                                   (wk_ref, bk_ref, k_ref),
                                   (wv_ref, bv_ref, v_ref)):
        o = jnp.dot(z, w_ref[:], preferred_element_type=jnp.float32) + bias_ref[:]
        o_ref[0] = o[:, :HW]
        o_ref[1] = o[:, HW:]


def _tc_proj(x, Wq, bq, Wk, bk, Wv, bv, g, b):
    wspec = pl.BlockSpec((EMBED, EMBED), lambda i: (0, 0))
    bspec = pl.BlockSpec((1, EMBED), lambda i: (0, 0))
    ospec = pl.BlockSpec((2, ROWS_BLK, HW), lambda i: (0, i, 0))
    oshape = jax.ShapeDtypeStruct((2, L, HW), jnp.float32)
    return pl.pallas_call(
        _proj_body,
        grid=(GRID,),
        in_specs=[pl.BlockSpec((ROWS_BLK, EMBED), lambda i: (i, 0)),
                  wspec, bspec, wspec, bspec, wspec, bspec, bspec, bspec],
        out_specs=[ospec, ospec, ospec],
        out_shape=[oshape, oshape, oshape],
    )(x, Wq, bq, Wk, bk, Wv, bv, g, b)


def _sc_edge_body(q_hbm, k_hbm, v_hbm, bias_hbm, row_hbm, col_hbm, out_hbm,
                  ridx, cidx, ridx2, cidx2, bbuf, qbuf, kbuf, vbuf, msgbuf,
                  acc_sh, sem):
    c = lax.axis_index("c")
    w = lax.axis_index("s")
    zeros16 = jnp.zeros((16,), jnp.float32)

    def _z(i, _):
        for j in range(ACC_W // 16):
            msgbuf[i, pl.ds(j * 16, 16)] = zeros16
        return 0
    lax.fori_loop(0, CHUNK, _z, 0)

    for j in range(STRIPE // SUBSTRIPE):
        pltpu.sync_copy(msgbuf.at[pl.ds(0, SUBSTRIPE)],
                        acc_sh.at[pl.ds(w * STRIPE + j * SUBSTRIPE, SUBSTRIPE)])
    plsc.subcore_barrier()

    coff = (c * L).astype(jnp.int32)
    nch = jnp.where(w < NCHUNK - (NCHUNK // NSUB) * NSUB,
                    NCHUNK // NSUB + 1, NCHUNK // NSUB)

    def _chunk(t, _):
        base = (w + t * NSUB) * CHUNK
        pltpu.sync_copy(row_hbm.at[pl.ds(base, CHUNK)], ridx)
        pltpu.sync_copy(col_hbm.at[pl.ds(base, CHUNK)], cidx)
        pltpu.sync_copy(bias_hbm.at[pl.ds(c * E + base, CHUNK)], bbuf)

        def _adj(i, _):
            ridx2[pl.ds(i * 16, 16)] = ridx[pl.ds(i * 16, 16)] + coff
            cidx2[pl.ds(i * 16, 16)] = cidx[pl.ds(i * 16, 16)] + coff
            return 0
        lax.fori_loop(0, CHUNK // 16, _adj, 0)

        cp_q = pltpu.async_copy(q_hbm.at[ridx2], qbuf, sem)
        cp_k = pltpu.async_copy(k_hbm.at[cidx2], kbuf, sem)
        cp_v = pltpu.async_copy(v_hbm.at[cidx2], vbuf, sem)
        cp_q.wait()
        cp_k.wait()
        cp_v.wait()

        def _grp(g, _):
            ebase = lax.iota(jnp.int32, 16) + g * 16
            for h in range(HH):
                dot = zeros16
                for d in range(DH):
                    colv = jnp.full((16,), h * DH + d, jnp.int32)
                    qv = plsc.load_gather(qbuf, [ebase, colv])
                    kv = plsc.load_gather(kbuf, [ebase, colv])
                    dot = dot + qv * kv
                bias = plsc.load_gather(bbuf, [ebase, jnp.full((16,), h, jnp.int32)])
                p = jnp.exp(dot * INV_SQRT_DH + bias)
                plsc.store_scatter(msgbuf, [ebase, jnp.full((16,), HW + h, jnp.int32)], p)
                for d in range(DH):
                    colv = jnp.full((16,), h * DH + d, jnp.int32)
                    vv = plsc.load_gather(vbuf, [ebase, colv])
                    plsc.store_scatter(msgbuf, [ebase, colv], p * vv)
            return 0
        lax.fori_loop(0, CHUNK // 16, _grp, 0)

        pltpu.sync_copy(msgbuf, acc_sh.at[ridx], add=True)
        return 0

    lax.fori_loop(0, nch, _chunk, 0)
    plsc.subcore_barrier()

    for j in range(STRIPE // SUBSTRIPE):
        r0 = w * STRIPE + j * SUBSTRIPE
        pltpu.sync_copy(acc_sh.at[pl.ds(r0, SUBSTRIPE)],
                        msgbuf.at[pl.ds(0, SUBSTRIPE)])
        pltpu.sync_copy(msgbuf.at[pl.ds(0, SUBSTRIPE)],
                        out_hbm.at[c, pl.ds(r0, SUBSTRIPE)])


def _sc_edge(qS, kS, vS, bias2, row32, col32):
    mesh = plsc.VectorSubcoreMesh(core_axis_name="c", subcore_axis_name="s",
                                  num_cores=2, num_subcores=NSUB)
    f = pl.kernel(
        _sc_edge_body,
        out_type=jax.ShapeDtypeStruct((2, L, ACC_W), jnp.float32),
        mesh=mesh,
        scratch_types=[
            pltpu.VMEM((CHUNK,), jnp.int32),
            pltpu.VMEM((CHUNK,), jnp.int32),
            pltpu.VMEM((CHUNK,), jnp.int32),
            pltpu.VMEM((CHUNK,), jnp.int32),
            pltpu.VMEM((CHUNK, HH), jnp.float32),
            pltpu.VMEM((CHUNK, HW), jnp.float32),
            pltpu.VMEM((CHUNK, HW), jnp.float32),
            pltpu.VMEM((CHUNK, HW), jnp.float32),
            pltpu.VMEM((CHUNK, ACC_W), jnp.float32),
            pltpu.VMEM_SHARED((L, ACC_W), jnp.float32),
            pltpu.SemaphoreType.DMA,
        ],
    )
    return f(qS, kS, vS, bias2, row32, col32)


def _finish_body(acc_ref, x_ref, wo_ref, bo_ref, g2_ref, b2_ref,
                 w1_ref, b1_ref, w2_ref, b2b_ref, out_ref):
    parts = []
    for half in range(2):
        accH = acc_ref[half]
        for h in range(HH):
            s = accH[:, HW + h:HW + h + 1] + 1e-9
            parts.append(accH[:, h * DH:(h + 1) * DH] / s)
    att = jnp.concatenate(parts, axis=1)
    y = x_ref[:] + jnp.dot(att, wo_ref[:], preferred_element_type=jnp.float32) + bo_ref[:]
    z = _ln(y, g2_ref[:], b2_ref[:])
    hmid = jnp.dot(z, w1_ref[:], preferred_element_type=jnp.float32) + b1_ref[:]
    act = hmid * jax.nn.sigmoid(hmid)
    out_ref[:] = y + jnp.dot(act, w2_ref[:], preferred_element_type=jnp.float32) + b2b_ref[:]


def _tc_finish(acc, x, Wo, bo, g2, b2, W1, b1, W2, b2b):
    return pl.pallas_call(
        _finish_body,
        grid=(GRID,),
        in_specs=[pl.BlockSpec((2, ROWS_BLK, ACC_W), lambda i: (0, i, 0)),
                  pl.BlockSpec((ROWS_BLK, EMBED), lambda i: (i, 0)),
                  pl.BlockSpec((EMBED, EMBED), lambda i: (0, 0)),
                  pl.BlockSpec((1, EMBED), lambda i: (0, 0)),
                  pl.BlockSpec((1, EMBED), lambda i: (0, 0)),
                  pl.BlockSpec((1, EMBED), lambda i: (0, 0)),
                  pl.BlockSpec((EMBED, HIDDEN), lambda i: (0, 0)),
                  pl.BlockSpec((1, HIDDEN), lambda i: (0, 0)),
                  pl.BlockSpec((HIDDEN, EMBED), lambda i: (0, 0)),
                  pl.BlockSpec((1, EMBED), lambda i: (0, 0))],
        out_specs=pl.BlockSpec((ROWS_BLK, EMBED), lambda i: (i, 0)),
        out_shape=jax.ShapeDtypeStruct((L, EMBED), jnp.float32),
    )(acc, x, Wo, bo, g2, b2, W1, b1, W2, b2b)


def kernel(x, row_index, col_index, att_bias, Wq, bq, Wk, bk, Wv, bv, Wo, bo,
           ln1_g, ln1_b, ln2_g, ln2_b, W1, b1, W2, b2):
    row32 = row_index.astype(jnp.int32)
    col32 = col_index.astype(jnp.int32)
    bias2 = jnp.stack([att_bias[:, :HH], att_bias[:, HH:]]).reshape(2 * E, HH)

    q2, k2, v2 = _tc_proj(x, Wq, bq.reshape(1, EMBED), Wk, bk.reshape(1, EMBED),
                          Wv, bv.reshape(1, EMBED), ln1_g.reshape(1, EMBED),
                          ln1_b.reshape(1, EMBED))
    qS = q2.reshape(2 * L, HW)
    kS = k2.reshape(2 * L, HW)
    vS = v2.reshape(2 * L, HW)

    acc = _sc_edge(qS, kS, vS, bias2, row32, col32)

    return _tc_finish(acc, x, Wo, bo.reshape(1, EMBED), ln2_g.reshape(1, EMBED),
                      ln2_b.reshape(1, EMBED), W1, b1.reshape(1, HIDDEN),
                      W2, b2.reshape(1, EMBED))


# speedup vs baseline: 5.0777x; 5.0777x over previous
import jax
import jax.numpy as jnp
from jax import lax
from jax.experimental import pallas as pl
from jax.experimental.pallas import tpu as pltpu
from jax.experimental.pallas import tpu_sc as plsc

EMBED = 256
HIDDEN = 1024
HEADS = 8
DH = EMBED // HEADS
HH = HEADS // 2
HW = EMBED // 2
L = 10000
E = 160000
INV_SQRT_DH = 1.0 / (DH ** 0.5)

ROWS_BLK = 1000
GRID = L // ROWS_BLK

NSUB = 16
CHUNK = 128
NCHUNK = E // CHUNK
NPHASES = 3
NPH = 3360
ACCM_ROWS = 3456
MSTRIPE = ACCM_ROWS // NSUB
MSUB = ((0, 128), (128, 88))
SROWS = 256
SSTRIPE = SROWS // NSUB
TRASH = NPH
STRASH = NPH // 32

FIN_BLK = 672
FIN_GRID = 15


def _ln(x, g, b, eps=1e-5):
    mu = jnp.mean(x, axis=-1, keepdims=True)
    var = jnp.mean((x - mu) ** 2, axis=-1, keepdims=True)
    return (x - mu) * lax.rsqrt(var + eps) * g + b


def _proj_body(x_ref, wq_ref, bq_ref, wk_ref, bk_ref, wv_ref, bv_ref,
               g_ref, b_ref, q_ref, k_ref, v_ref):
    z = _ln(x_ref[:], g_ref[:], b_ref[:])
    for w_ref, bias_ref, o_ref in ((wq_ref, bq_ref, q_ref),
                                   (wk_ref, bk_ref, k_ref),
                                   (wv_ref, bv_ref, v_ref)):
        o = jnp.dot(z, w_ref[:], preferred_element_type=jnp.float32) + bias_ref[:]
        o_ref[0] = o[:, :HW]
        o_ref[1] = o[:, HW:]


def _tc_proj(x, Wq, bq, Wk, bk, Wv, bv, g, b):
    wspec = pl.BlockSpec((EMBED, EMBED), lambda i: (0, 0))
    bspec = pl.BlockSpec((1, EMBED), lambda i: (0, 0))
    ospec = pl.BlockSpec((2, ROWS_BLK, HW), lambda i: (0, i, 0))
    oshape = jax.ShapeDtypeStruct((2, L, HW), jnp.float32)
    return pl.pallas_call(
        _proj_body,
        grid=(GRID,),
        in_specs=[pl.BlockSpec((ROWS_BLK, EMBED), lambda i: (i, 0)),
                  wspec, bspec, wspec, bspec, wspec, bspec, bspec, bspec],
        out_specs=[ospec, ospec, ospec],
        out_shape=[oshape, oshape, oshape],
    )(x, Wq, bq, Wk, bk, Wv, bv, g, b)


def _sc_edge_body(q_hbm, k_hbm, v_hbm, bias_hbm, row_hbm, col_hbm,
                  outm_hbm, outs_hbm, dumpm_hbm, dumps_hbm,
                  ridx, cidx, ridx2, cidx2, lidx, sidx, bbuf, qbuf, kbuf,
                  vbuf, msgbuf, sbuf, accm, accs, sem):
    c = lax.axis_index("c")
    w = lax.axis_index("s")
    zeros16 = jnp.zeros((16,), jnp.float32)

    def _z(i, _):
        for j in range(HW // 16):
            msgbuf[i, pl.ds(j * 16, 16)] = zeros16
            sbuf[i, pl.ds(j * 16, 16)] = zeros16
        return 0
    lax.fori_loop(0, CHUNK, _z, 0)

    coff = (c * L).astype(jnp.int32)
    nch = jnp.where(w < NCHUNK - (NCHUNK // NSUB) * NSUB,
                    NCHUNK // NSUB + 1, NCHUNK // NSUB)

    def _phase(ph, _):
        def _zm(i, _2):
            for j in range(HW // 16):
                msgbuf[i, pl.ds(j * 16, 16)] = zeros16
            return 0
        lax.fori_loop(0, CHUNK, _zm, 0)
        for off, ln in MSUB:
            pltpu.sync_copy(msgbuf.at[pl.ds(0, ln)],
                            accm.at[pl.ds(w * MSTRIPE + off, ln)])
        pltpu.sync_copy(msgbuf.at[pl.ds(0, SSTRIPE)],
                        accs.at[pl.ds(w * SSTRIPE, SSTRIPE)])
        plsc.subcore_barrier()

        ph0 = (ph * NPH).astype(jnp.int32)

        def _chunk(t, _):
            base = (w + t * NSUB) * CHUNK
            pltpu.sync_copy(row_hbm.at[pl.ds(base, CHUNK)], ridx)

            def _adj(i, _):
                rv = ridx[pl.ds(i * 16, 16)]
                local = rv - ph0
                inh = (local >= 0) & (local < NPH)
                lidx[pl.ds(i * 16, 16)] = jnp.where(inh, local, TRASH)
                sidx[pl.ds(i * 16, 16)] = jnp.where(
                    inh, lax.shift_right_logical(local, 5), STRASH)
                return 0
            lax.fori_loop(0, CHUNK // 16, _adj, 0)

            @pl.when(ph == 0)
            def _compute():
                pltpu.sync_copy(col_hbm.at[pl.ds(base, CHUNK)], cidx)
                pltpu.sync_copy(bias_hbm.at[pl.ds(c * E + base, CHUNK)], bbuf)

                def _adj2(i, _):
                    ridx2[pl.ds(i * 16, 16)] = ridx[pl.ds(i * 16, 16)] + coff
                    cidx2[pl.ds(i * 16, 16)] = cidx[pl.ds(i * 16, 16)] + coff
                    return 0
                lax.fori_loop(0, CHUNK // 16, _adj2, 0)

                cp_q = pltpu.async_copy(q_hbm.at[ridx2], qbuf, sem)
                cp_k = pltpu.async_copy(k_hbm.at[cidx2], kbuf, sem)
                cp_v = pltpu.async_copy(v_hbm.at[cidx2], vbuf, sem)
                cp_q.wait()
                cp_k.wait()
                cp_v.wait()

                @plsc.parallel_loop(0, CHUNK // 16, unroll=4)
                def _grp(g):
                    ebase = lax.iota(jnp.int32, 16) + g * 16
                    cell0 = (ridx[pl.ds(g * 16, 16)] & 31) * HH
                    for h in range(HH):
                        dot = zeros16
                        for d in range(DH):
                            colv = jnp.full((16,), h * DH + d, jnp.int32)
                            qv = plsc.load_gather(qbuf, [ebase, colv])
                            kv = plsc.load_gather(kbuf, [ebase, colv])
                            dot = dot + qv * kv
                        bias = plsc.load_gather(
                            bbuf, [ebase, jnp.full((16,), h, jnp.int32)])
                        p = jnp.exp(dot * INV_SQRT_DH + bias)
                        plsc.store_scatter(sbuf, [ebase, cell0 + h], p)
                        for d in range(DH):
                            colv = jnp.full((16,), h * DH + d, jnp.int32)
                            vv = plsc.load_gather(vbuf, [ebase, colv])
                            plsc.store_scatter(msgbuf, [ebase, colv], p * vv)

                pltpu.sync_copy(msgbuf, dumpm_hbm.at[c, pl.ds(base, CHUNK)])
                pltpu.sync_copy(sbuf, dumps_hbm.at[c, pl.ds(base, CHUNK)])

            @pl.when(ph != 0)
            def _reload():
                pltpu.sync_copy(dumpm_hbm.at[c, pl.ds(base, CHUNK)], msgbuf)
                pltpu.sync_copy(dumps_hbm.at[c, pl.ds(base, CHUNK)], sbuf)

            pltpu.sync_copy(msgbuf, accm.at[lidx], add=True)
            pltpu.sync_copy(sbuf, accs.at[sidx], add=True)

            @pl.when(ph == 0)
            def _zs_all():
                def _zs(g, _):
                    ebase = lax.iota(jnp.int32, 16) + g * 16
                    cell0 = (ridx[pl.ds(g * 16, 16)] & 31) * HH
                    for h in range(HH):
                        plsc.store_scatter(sbuf, [ebase, cell0 + h], zeros16)
                    return 0
                lax.fori_loop(0, CHUNK // 16, _zs, 0)
            return 0

        lax.fori_loop(0, nch, _chunk, 0)
        plsc.subcore_barrier()

        for off, ln in MSUB:
            r0 = w * MSTRIPE + off
            pltpu.sync_copy(accm.at[pl.ds(r0, ln)], msgbuf.at[pl.ds(0, ln)])
            pltpu.sync_copy(msgbuf.at[pl.ds(0, ln)],
                            outm_hbm.at[c, ph, pl.ds(r0, ln)])
        s0 = w * SSTRIPE
        pltpu.sync_copy(accs.at[pl.ds(s0, SSTRIPE)], msgbuf.at[pl.ds(0, SSTRIPE)])
        pltpu.sync_copy(msgbuf.at[pl.ds(0, SSTRIPE)],
                        outs_hbm.at[c, ph, pl.ds(s0, SSTRIPE)])
        return 0

    lax.fori_loop(0, NPHASES, _phase, 0)


def _sc_edge(qS, kS, vS, bias2, row32, col32):
    mesh = plsc.VectorSubcoreMesh(core_axis_name="c", subcore_axis_name="s",
                                  num_cores=2, num_subcores=NSUB)
    f = pl.kernel(
        _sc_edge_body,
        out_type=[jax.ShapeDtypeStruct((2, NPHASES, ACCM_ROWS, HW), jnp.float32),
                  jax.ShapeDtypeStruct((2, NPHASES, SROWS, HW), jnp.float32),
                  jax.ShapeDtypeStruct((2, E, HW), jnp.float32),
                  jax.ShapeDtypeStruct((2, E, HW), jnp.float32)],
        mesh=mesh,
        scratch_types=[
            pltpu.VMEM((CHUNK,), jnp.int32),
            pltpu.VMEM((CHUNK,), jnp.int32),
            pltpu.VMEM((CHUNK,), jnp.int32),
            pltpu.VMEM((CHUNK,), jnp.int32),
            pltpu.VMEM((CHUNK,), jnp.int32),
            pltpu.VMEM((CHUNK,), jnp.int32),
            pltpu.VMEM((CHUNK, HH), jnp.float32),
            pltpu.VMEM((CHUNK, HW), jnp.float32),
            pltpu.VMEM((CHUNK, HW), jnp.float32),
            pltpu.VMEM((CHUNK, HW), jnp.float32),
            pltpu.VMEM((CHUNK, HW), jnp.float32),
            pltpu.VMEM((CHUNK, HW), jnp.float32),
            pltpu.VMEM_SHARED((ACCM_ROWS, HW), jnp.float32),
            pltpu.VMEM_SHARED((SROWS, HW), jnp.float32),
            pltpu.SemaphoreType.DMA,
        ],
        compiler_params=pltpu.CompilerParams(needs_layout_passes=False, disable_bounds_checks=True),
    )
    return f(qS, kS, vS, bias2, row32, col32)[:2]


def _finish_body(accm_ref, s_ref, x_ref, wo_ref, bo_ref, g2_ref, b2_ref,
                 w1_ref, b1_ref, w2_ref, b2b_ref, out_ref):
    parts = []
    for half in range(2):
        accH = accm_ref[half, 0]
        sH = s_ref[half, 0]
        for h in range(HH):
            s = sH[:, h:h + 1] + 1e-9
            parts.append(accH[:, h * DH:(h + 1) * DH] / s)
    att = jnp.concatenate(parts, axis=1)
    y = x_ref[:] + jnp.dot(att, wo_ref[:], preferred_element_type=jnp.float32) + bo_ref[:]
    z = _ln(y, g2_ref[:], b2_ref[:])
    hmid = jnp.dot(z, w1_ref[:], preferred_element_type=jnp.float32) + b1_ref[:]
    act = hmid * jax.nn.sigmoid(hmid)
    out_ref[:] = y + jnp.dot(act, w2_ref[:], preferred_element_type=jnp.float32) + b2b_ref[:]


def _tc_finish(accm, s2, x, Wo, bo, g2, b2, W1, b1, W2, b2b):
    return pl.pallas_call(
        _finish_body,
        grid=(FIN_GRID,),
        in_specs=[pl.BlockSpec((2, 1, FIN_BLK, HW),
                               lambda i: (0, i // 5, i % 5, 0)),
                  pl.BlockSpec((2, 1, FIN_BLK, HH),
                               lambda i: (0, i // 5, i % 5, 0)),
                  pl.BlockSpec((FIN_BLK, EMBED), lambda i: (i, 0)),
                  pl.BlockSpec((EMBED, EMBED), lambda i: (0, 0)),
                  pl.BlockSpec((1, EMBED), lambda i: (0, 0)),
                  pl.BlockSpec((1, EMBED), lambda i: (0, 0)),
                  pl.BlockSpec((1, EMBED), lambda i: (0, 0)),
                  pl.BlockSpec((EMBED, HIDDEN), lambda i: (0, 0)),
                  pl.BlockSpec((1, HIDDEN), lambda i: (0, 0)),
                  pl.BlockSpec((HIDDEN, EMBED), lambda i: (0, 0)),
                  pl.BlockSpec((1, EMBED), lambda i: (0, 0))],
        out_specs=pl.BlockSpec((FIN_BLK, EMBED), lambda i: (i, 0)),
        out_shape=jax.ShapeDtypeStruct((L, EMBED), jnp.float32),
    )(accm, s2, x, Wo, bo, g2, b2, W1, b1, W2, b2b)


def kernel(x, row_index, col_index, att_bias, Wq, bq, Wk, bk, Wv, bv, Wo, bo,
           ln1_g, ln1_b, ln2_g, ln2_b, W1, b1, W2, b2):
    row32 = row_index.astype(jnp.int32)
    col32 = col_index.astype(jnp.int32)
    bias2 = jnp.stack([att_bias[:, :HH], att_bias[:, HH:]]).reshape(2 * E, HH)

    q2, k2, v2 = _tc_proj(x, Wq, bq.reshape(1, EMBED), Wk, bk.reshape(1, EMBED),
                          Wv, bv.reshape(1, EMBED), ln1_g.reshape(1, EMBED),
                          ln1_b.reshape(1, EMBED))
    qS = q2.reshape(2 * L, HW)
    kS = k2.reshape(2 * L, HW)
    vS = v2.reshape(2 * L, HW)

    accm, accs = _sc_edge(qS, kS, vS, bias2, row32, col32)
    s2 = accs.reshape(2, NPHASES, SROWS * 32, HH)

    return _tc_finish(accm, s2, x, Wo, bo.reshape(1, EMBED),
                      ln2_g.reshape(1, EMBED), ln2_b.reshape(1, EMBED),
                      W1, b1.reshape(1, HIDDEN), W2, b2.reshape(1, EMBED))


# speedup vs baseline: 5.1389x; 1.0121x over previous
import jax
import jax.numpy as jnp
from jax import lax
from jax.experimental import pallas as pl
from jax.experimental.pallas import tpu as pltpu
from jax.experimental.pallas import tpu_sc as plsc

EMBED = 256
HIDDEN = 1024
HEADS = 8
DH = EMBED // HEADS
HH = HEADS // 2
HW = EMBED // 2
L = 10000
E = 160000
INV_SQRT_DH = 1.0 / (DH ** 0.5)

ROWS_BLK = 1000
GRID = L // ROWS_BLK

NSUB = 16
CHUNK = 128
NCHUNK = E // CHUNK
NPHASES = 3
NPH = 3360
ACCM_ROWS = 3456
MSTRIPE = ACCM_ROWS // NSUB
MSUB = ((0, 128), (128, 88))
SROWS = 256
SSTRIPE = SROWS // NSUB
TRASH = NPH
STRASH = NPH // 32

FIN_BLK = 672
FIN_GRID = 15


def _ln(x, g, b, eps=1e-5):
    mu = jnp.mean(x, axis=-1, keepdims=True)
    var = jnp.mean((x - mu) ** 2, axis=-1, keepdims=True)
    return (x - mu) * lax.rsqrt(var + eps) * g + b


def _proj_body(x_ref, wq_ref, bq_ref, wk_ref, bk_ref, wv_ref, bv_ref,
               g_ref, b_ref, q_ref, k_ref, v_ref):
    z = _ln(x_ref[:], g_ref[:], b_ref[:])
    for w_ref, bias_ref, o_ref in ((wq_ref, bq_ref, q_ref),
                                   (wk_ref, bk_ref, k_ref),
                                   (wv_ref, bv_ref, v_ref)):
        o = jnp.dot(z, w_ref[:], preferred_element_type=jnp.float32) + bias_ref[:]
        o_ref[0] = o[:, :HW]
        o_ref[1] = o[:, HW:]


def _tc_proj(x, Wq, bq, Wk, bk, Wv, bv, g, b):
    wspec = pl.BlockSpec((EMBED, EMBED), lambda i: (0, 0))
    bspec = pl.BlockSpec((1, EMBED), lambda i: (0, 0))
    ospec = pl.BlockSpec((2, ROWS_BLK, HW), lambda i: (0, i, 0))
    oshape = jax.ShapeDtypeStruct((2, L, HW), jnp.float32)
    return pl.pallas_call(
        _proj_body,
        grid=(GRID,),
        in_specs=[pl.BlockSpec((ROWS_BLK, EMBED), lambda i: (i, 0)),
                  wspec, bspec, wspec, bspec, wspec, bspec, bspec, bspec],
        out_specs=[ospec, ospec, ospec],
        out_shape=[oshape, oshape, oshape],
    )(x, Wq, bq, Wk, bk, Wv, bv, g, b)


def _sc_edge_body(q_hbm, k_hbm, v_hbm, bias_hbm, row_hbm, col_hbm,
                  outm_hbm, outs_hbm, dumpm_hbm, dumps_hbm,
                  ridx, cidx, ridx2, cidx2, lidx, sidx, bbuf, qbuf, kbuf,
                  vbuf, msgbuf, sbuf, accm, accs, sem):
    c = lax.axis_index("c")
    w = lax.axis_index("s")
    zeros16 = jnp.zeros((16,), jnp.float32)

    def _z(i, _):
        for j in range(HW // 16):
            msgbuf[i, pl.ds(j * 16, 16)] = zeros16
            sbuf[i, pl.ds(j * 16, 16)] = zeros16
        return 0
    lax.fori_loop(0, CHUNK, _z, 0)

    coff = (c * L).astype(jnp.int32)
    nch = jnp.where(w < NCHUNK - (NCHUNK // NSUB) * NSUB,
                    NCHUNK // NSUB + 1, NCHUNK // NSUB)

    def _phase(ph, _):
        def _zm(i, _2):
            for j in range(HW // 16):
                msgbuf[i, pl.ds(j * 16, 16)] = zeros16
            return 0
        lax.fori_loop(0, CHUNK, _zm, 0)
        for off, ln in MSUB:
            pltpu.sync_copy(msgbuf.at[pl.ds(0, ln)],
                            accm.at[pl.ds(w * MSTRIPE + off, ln)])
        pltpu.sync_copy(msgbuf.at[pl.ds(0, SSTRIPE)],
                        accs.at[pl.ds(w * SSTRIPE, SSTRIPE)])
        plsc.subcore_barrier()

        ph0 = (ph * NPH).astype(jnp.int32)

        def _chunk(t, _):
            base = (w + t * NSUB) * CHUNK
            pltpu.sync_copy(row_hbm.at[pl.ds(base, CHUNK)], ridx)

            def _adj(i, _):
                rv = ridx[pl.ds(i * 16, 16)]
                local = rv - ph0
                inh = (local >= 0) & (local < NPH)
                lidx[pl.ds(i * 16, 16)] = jnp.where(inh, local, TRASH)
                sidx[pl.ds(i * 16, 16)] = jnp.where(
                    inh, lax.shift_right_logical(local, 5), STRASH)
                return 0
            lax.fori_loop(0, CHUNK // 16, _adj, 0)

            @pl.when(ph == 0)
            def _compute():
                pltpu.sync_copy(col_hbm.at[pl.ds(base, CHUNK)], cidx)
                pltpu.sync_copy(bias_hbm.at[pl.ds(c * E + base, CHUNK)], bbuf)

                def _adj2(i, _):
                    ridx2[pl.ds(i * 16, 16)] = ridx[pl.ds(i * 16, 16)] + coff
                    cidx2[pl.ds(i * 16, 16)] = cidx[pl.ds(i * 16, 16)] + coff
                    return 0
                lax.fori_loop(0, CHUNK // 16, _adj2, 0)

                cp_q = pltpu.async_copy(q_hbm.at[ridx2], qbuf, sem)
                cp_k = pltpu.async_copy(k_hbm.at[cidx2], kbuf, sem)
                cp_v = pltpu.async_copy(v_hbm.at[cidx2], vbuf, sem)
                cp_q.wait()
                cp_k.wait()
                cp_v.wait()

                @plsc.parallel_loop(0, CHUNK // 16, unroll=4)
                def _grp(g):
                    ebase = lax.iota(jnp.int32, 16) + g * 16
                    cell0 = (ridx[pl.ds(g * 16, 16)] & 31) * HH
                    for h in range(HH):
                        dot_a = zeros16
                        dot_b = zeros16
                        for d in range(0, DH, 2):
                            colv = jnp.full((16,), h * DH + d, jnp.int32)
                            colw = jnp.full((16,), h * DH + d + 1, jnp.int32)
                            qa = plsc.load_gather(qbuf, [ebase, colv])
                            ka = plsc.load_gather(kbuf, [ebase, colv])
                            qb = plsc.load_gather(qbuf, [ebase, colw])
                            kb = plsc.load_gather(kbuf, [ebase, colw])
                            dot_a = dot_a + qa * ka
                            dot_b = dot_b + qb * kb
                        dot = dot_a + dot_b
                        bias = plsc.load_gather(
                            bbuf, [ebase, jnp.full((16,), h, jnp.int32)])
                        p = jnp.exp(dot * INV_SQRT_DH + bias)
                        plsc.store_scatter(sbuf, [ebase, cell0 + h], p)
                        for d in range(DH):
                            colv = jnp.full((16,), h * DH + d, jnp.int32)
                            vv = plsc.load_gather(vbuf, [ebase, colv])
                            plsc.store_scatter(msgbuf, [ebase, colv], p * vv)

                pltpu.sync_copy(msgbuf, dumpm_hbm.at[c, pl.ds(base, CHUNK)])
                pltpu.sync_copy(sbuf, dumps_hbm.at[c, pl.ds(base, CHUNK)])

            @pl.when(ph != 0)
            def _reload():
                pltpu.sync_copy(dumpm_hbm.at[c, pl.ds(base, CHUNK)], msgbuf)
                pltpu.sync_copy(dumps_hbm.at[c, pl.ds(base, CHUNK)], sbuf)

            pltpu.sync_copy(msgbuf, accm.at[lidx], add=True)
            pltpu.sync_copy(sbuf, accs.at[sidx], add=True)

            @pl.when(ph == 0)
            def _zs_all():
                def _zs(g, _):
                    ebase = lax.iota(jnp.int32, 16) + g * 16
                    cell0 = (ridx[pl.ds(g * 16, 16)] & 31) * HH
                    for h in range(HH):
                        plsc.store_scatter(sbuf, [ebase, cell0 + h], zeros16)
                    return 0
                lax.fori_loop(0, CHUNK // 16, _zs, 0)
            return 0

        lax.fori_loop(0, nch, _chunk, 0)
        plsc.subcore_barrier()

        for off, ln in MSUB:
            r0 = w * MSTRIPE + off
            pltpu.sync_copy(accm.at[pl.ds(r0, ln)], msgbuf.at[pl.ds(0, ln)])
            pltpu.sync_copy(msgbuf.at[pl.ds(0, ln)],
                            outm_hbm.at[c, ph, pl.ds(r0, ln)])
        s0 = w * SSTRIPE
        pltpu.sync_copy(accs.at[pl.ds(s0, SSTRIPE)], msgbuf.at[pl.ds(0, SSTRIPE)])
        pltpu.sync_copy(msgbuf.at[pl.ds(0, SSTRIPE)],
                        outs_hbm.at[c, ph, pl.ds(s0, SSTRIPE)])
        return 0

    lax.fori_loop(0, NPHASES, _phase, 0)


def _sc_edge(qS, kS, vS, bias2, row32, col32):
    mesh = plsc.VectorSubcoreMesh(core_axis_name="c", subcore_axis_name="s",
                                  num_cores=2, num_subcores=NSUB)
    f = pl.kernel(
        _sc_edge_body,
        out_type=[jax.ShapeDtypeStruct((2, NPHASES, ACCM_ROWS, HW), jnp.float32),
                  jax.ShapeDtypeStruct((2, NPHASES, SROWS, HW), jnp.float32),
                  jax.ShapeDtypeStruct((2, E, HW), jnp.float32),
                  jax.ShapeDtypeStruct((2, E, HW), jnp.float32)],
        mesh=mesh,
        scratch_types=[
            pltpu.VMEM((CHUNK,), jnp.int32),
            pltpu.VMEM((CHUNK,), jnp.int32),
            pltpu.VMEM((CHUNK,), jnp.int32),
            pltpu.VMEM((CHUNK,), jnp.int32),
            pltpu.VMEM((CHUNK,), jnp.int32),
            pltpu.VMEM((CHUNK,), jnp.int32),
            pltpu.VMEM((CHUNK, HH), jnp.float32),
            pltpu.VMEM((CHUNK, HW), jnp.float32),
            pltpu.VMEM((CHUNK, HW), jnp.float32),
            pltpu.VMEM((CHUNK, HW), jnp.float32),
            pltpu.VMEM((CHUNK, HW), jnp.float32),
            pltpu.VMEM((CHUNK, HW), jnp.float32),
            pltpu.VMEM_SHARED((ACCM_ROWS, HW), jnp.float32),
            pltpu.VMEM_SHARED((SROWS, HW), jnp.float32),
            pltpu.SemaphoreType.DMA,
        ],
        compiler_params=pltpu.CompilerParams(needs_layout_passes=False, disable_bounds_checks=True),
    )
    return f(qS, kS, vS, bias2, row32, col32)[:2]


def _finish_body(accm_ref, s_ref, x_ref, wo_ref, bo_ref, g2_ref, b2_ref,
                 w1_ref, b1_ref, w2_ref, b2b_ref, out_ref):
    parts = []
    for half in range(2):
        accH = accm_ref[half, 0]
        sH = s_ref[half, 0]
        for h in range(HH):
            s = sH[:, h:h + 1] + 1e-9
            parts.append(accH[:, h * DH:(h + 1) * DH] / s)
    att = jnp.concatenate(parts, axis=1)
    y = x_ref[:] + jnp.dot(att, wo_ref[:], preferred_element_type=jnp.float32) + bo_ref[:]
    z = _ln(y, g2_ref[:], b2_ref[:])
    hmid = jnp.dot(z, w1_ref[:], preferred_element_type=jnp.float32) + b1_ref[:]
    act = hmid * jax.nn.sigmoid(hmid)
    out_ref[:] = y + jnp.dot(act, w2_ref[:], preferred_element_type=jnp.float32) + b2b_ref[:]


def _tc_finish(accm, s2, x, Wo, bo, g2, b2, W1, b1, W2, b2b):
    return pl.pallas_call(
        _finish_body,
        grid=(FIN_GRID,),
        in_specs=[pl.BlockSpec((2, 1, FIN_BLK, HW),
                               lambda i: (0, i // 5, i % 5, 0)),
                  pl.BlockSpec((2, 1, FIN_BLK, HH),
                               lambda i: (0, i // 5, i % 5, 0)),
                  pl.BlockSpec((FIN_BLK, EMBED), lambda i: (i, 0)),
                  pl.BlockSpec((EMBED, EMBED), lambda i: (0, 0)),
                  pl.BlockSpec((1, EMBED), lambda i: (0, 0)),
                  pl.BlockSpec((1, EMBED), lambda i: (0, 0)),
                  pl.BlockSpec((1, EMBED), lambda i: (0, 0)),
                  pl.BlockSpec((EMBED, HIDDEN), lambda i: (0, 0)),
                  pl.BlockSpec((1, HIDDEN), lambda i: (0, 0)),
                  pl.BlockSpec((HIDDEN, EMBED), lambda i: (0, 0)),
                  pl.BlockSpec((1, EMBED), lambda i: (0, 0))],
        out_specs=pl.BlockSpec((FIN_BLK, EMBED), lambda i: (i, 0)),
        out_shape=jax.ShapeDtypeStruct((L, EMBED), jnp.float32),
    )(accm, s2, x, Wo, bo, g2, b2, W1, b1, W2, b2b)


def kernel(x, row_index, col_index, att_bias, Wq, bq, Wk, bk, Wv, bv, Wo, bo,
           ln1_g, ln1_b, ln2_g, ln2_b, W1, b1, W2, b2):
    row32 = row_index.astype(jnp.int32)
    col32 = col_index.astype(jnp.int32)
    bias2 = jnp.stack([att_bias[:, :HH], att_bias[:, HH:]]).reshape(2 * E, HH)

    q2, k2, v2 = _tc_proj(x, Wq, bq.reshape(1, EMBED), Wk, bk.reshape(1, EMBED),
                          Wv, bv.reshape(1, EMBED), ln1_g.reshape(1, EMBED),
                          ln1_b.reshape(1, EMBED))
    qS = q2.reshape(2 * L, HW)
    kS = k2.reshape(2 * L, HW)
    vS = v2.reshape(2 * L, HW)

    accm, accs = _sc_edge(qS, kS, vS, bias2, row32, col32)
    s2 = accs.reshape(2, NPHASES, SROWS * 32, HH)

    return _tc_finish(accm, s2, x, Wo, bo.reshape(1, EMBED),
                      ln2_g.reshape(1, EMBED), ln2_b.reshape(1, EMBED),
                      W1, b1.reshape(1, HIDDEN), W2, b2.reshape(1, EMBED))
